# baseline (speedup 1.0000x reference)
import functools

import jax
import jax.numpy as jnp
from jax import lax
from jax.experimental import pallas as pl
from jax.experimental.pallas import tpu as pltpu
from jax.experimental.pallas import tpu_sc as plsc

C, H, W = 9, 180, 240
B, N = 16, 125000
NEG = 0.1

NP = 131072
CN = 8192
VOX_PER_B = 2 * C * H * W
NUM_VOX = VOX_PER_B * B

NS = 16
ROW = 128
ROWS_PER_CHUNK = 16
CHUNK = ROW * ROWS_PER_CHUNK
PAIRS_PER_BATCH = C * NP
PAIRS_PER_TEC = PAIRS_PER_BATCH // NS
CHUNKS_PER_TEC = PAIRS_PER_TEC // CHUNK
CHUNKS_PER_BATCH = PAIRS_PER_BATCH // CHUNK
ACC_PAD = 786432
ZSTRIPE = ACC_PAD // NS
ZBUF = 8192
OSTRIPE = VOX_PER_B // NS
OPIECES = [(i * ZBUF, ZBUF) for i in range(OSTRIPE // ZBUF)]
OPIECES.append((OSTRIPE - OSTRIPE % ZBUF, OSTRIPE % ZBUF))


def _tc_body(ev_ref, w1_ref, b1a_ref, w2_ref, b2_ref, w3_ref, b3_ref,
             idx_ref, val_ref):
    ev = ev_ref[0]
    x = ev[0:1, :]
    y = ev[1:2, :]
    t = ev[2:3, :]
    p = ev[3:4, :]
    b = pl.program_id(0)
    bf = b.astype(jnp.float32)
    p2 = (p + 1.0) / 2.0
    idx_before = ((x + 240.0 * y) + 388800.0 * p2) + 777600.0 * bf
    w1 = w1_ref[...]
    w2 = w2_ref[...]
    b2 = b2_ref[...]
    w3 = w3_ref[...]
    b3 = b3_ref[...]
    q = w1 * t
    for i in range(C):
        h1 = q + b1a_ref[:, i:i + 1]
        h1 = jnp.maximum(h1, NEG * h1)
        h2 = jnp.dot(w2, h1, preferred_element_type=jnp.float32) + b2
        h2 = jnp.maximum(h2, NEG * h2)
        h3 = jnp.dot(w3, h2, preferred_element_type=jnp.float32) + b3
        val = t * h3
        idxf = idx_before + float(W * H * i)
        idxg = jnp.clip(idxf, 0.0, float(NUM_VOX - 1)).astype(jnp.int32)
        lidx = idxg - b * VOX_PER_B
        lidx = jnp.clip(lidx, 0, VOX_PER_B - 1)
        idx_ref[0, i] = lidx[0]
        val_ref[0, i] = val[0]


def _tc_stage(ev_t, w1, b1, w2, b2, w3, b3):
    grid = (B, NP // CN)
    wspec = lambda shape: pl.BlockSpec(shape, lambda bb, nn: (0, 0))
    return pl.pallas_call(
        _tc_body,
        grid=grid,
        in_specs=[
            pl.BlockSpec((1, 4, CN), lambda bb, nn: (bb, 0, nn)),
            wspec((16, 1)), wspec((16, C)), wspec((16, 16)),
            wspec((16, 1)), wspec((1, 16)), wspec((1, 1)),
        ],
        out_specs=[
            pl.BlockSpec((1, C, CN), lambda bb, nn: (bb, 0, nn)),
            pl.BlockSpec((1, C, CN), lambda bb, nn: (bb, 0, nn)),
        ],
        out_shape=[
            jax.ShapeDtypeStruct((B, C, NP), jnp.int32),
            jax.ShapeDtypeStruct((B, C, NP), jnp.float32),
        ],
    )(ev_t, w1, b1, w2, b2, w3, b3)


def _sc_body(idx_hbm, val_hbm, out_hbm, acc, zbuf, idx_buf, val_buf, obuf, sem):
    c = lax.axis_index("c")
    s = lax.axis_index("s")

    def _fill(j, _):
        zbuf[pl.ds(j * 16, 16)] = jnp.zeros((16,), jnp.float32)
        return 0
    lax.fori_loop(0, ZBUF // 16, _fill, 0)

    def batch_body(bi, _):
        b = c * (B // 2) + bi
        def zero_body(z, _):
            off = pl.multiple_of(s * ZSTRIPE + z * ZBUF, 8)
            pltpu.sync_copy(zbuf, acc.at[pl.ds(off, ZBUF)])
            return 0
        lax.fori_loop(0, ZSTRIPE // ZBUF, zero_body, 0)
        plsc.subcore_barrier()

        base_cid = b * CHUNKS_PER_BATCH + s * CHUNKS_PER_TEC

        def chunk_body(ch, _):
            cid = base_cid + ch
            pltpu.sync_copy(idx_hbm.at[cid], idx_buf)
            pltpu.sync_copy(val_hbm.at[cid], val_buf)
            cps = []
            for r in range(ROWS_PER_CHUNK):
                cps.append(pltpu.async_copy(
                    val_buf.at[r], acc.at[idx_buf.at[r]], sem, add=True))
            for cp in cps:
                cp.wait()
            return 0
        lax.fori_loop(0, CHUNKS_PER_TEC, chunk_body, 0)
        plsc.subcore_barrier()

        for poff, plen in OPIECES:
            src_off = pl.multiple_of(s * OSTRIPE + poff, 8)
            dst_off = pl.multiple_of(b * VOX_PER_B + s * OSTRIPE + poff, 8)
            pltpu.sync_copy(acc.at[pl.ds(src_off, plen)],
                            obuf.at[pl.ds(0, plen)])
            pltpu.sync_copy(obuf.at[pl.ds(0, plen)],
                            out_hbm.at[pl.ds(dst_off, plen)])
        return 0

    lax.fori_loop(0, B // 2, batch_body, 0)


def _sc_scatter(idx3, val3):
    fn = pl.kernel(
        _sc_body,
        out_type=jax.ShapeDtypeStruct((NUM_VOX,), jnp.float32),
        mesh=plsc.VectorSubcoreMesh(core_axis_name="c", subcore_axis_name="s"),
        scratch_types=[
            pltpu.VMEM_SHARED((ACC_PAD,), jnp.float32),
            pltpu.VMEM((ZBUF,), jnp.float32),
            pltpu.VMEM((ROWS_PER_CHUNK, ROW), jnp.int32),
            pltpu.VMEM((ROWS_PER_CHUNK, ROW), jnp.float32),
            pltpu.VMEM((ZBUF,), jnp.float32),
            pltpu.SemaphoreType.DMA,
        ],
    )
    return fn(idx3, val3)


@jax.jit
def _run(events, w1, b1, w2, b2, w3, b3):
    ev_t = jnp.transpose(events, (0, 2, 1))
    ev_t = jnp.pad(ev_t, ((0, 0), (0, 0), (0, NP - N)))
    w1c = w1.reshape(16, 1)
    bins = (jnp.arange(C, dtype=jnp.float32) / (C - 1)).reshape(1, C)
    b1a = b1.reshape(16, 1) - w1c * bins
    idx, val = _tc_stage(
        ev_t, w1c, b1a, w2,
        b2.reshape(16, 1), w3.reshape(1, 16), b3.reshape(1, 1),
    )
    idx3 = idx.reshape(B * CHUNKS_PER_BATCH, ROWS_PER_CHUNK, ROW)
    val3 = val.reshape(B * CHUNKS_PER_BATCH, ROWS_PER_CHUNK, ROW)
    vox = _sc_scatter(idx3, val3)
    return vox.reshape(B, 2 * C, H, W)


def kernel(events_list, device, w1, b1, w2, b2, w3, b3):
    return _run(events_list, w1, b1, w2, b2, w3, b3)


# speedup vs baseline: 1.0982x; 1.0982x over previous
import jax
import jax.numpy as jnp
from jax import lax
from jax.experimental import pallas as pl
from jax.experimental.pallas import tpu as pltpu
from jax.experimental.pallas import tpu_sc as plsc

C, H, W = 9, 180, 240
B, N = 16, 125000
NEG = 0.1

NP = 131072
CN = 8192
VOX_PER_B = 2 * C * H * W
NUM_VOX = VOX_PER_B * B

NS = 16
ROW = 128
ROWS_PER_CHUNK = 16
CHUNK = ROW * ROWS_PER_CHUNK
PAIRS_PER_BATCH = C * NP
PAIRS_PER_TEC = PAIRS_PER_BATCH // NS
CHUNKS_PER_TEC = PAIRS_PER_TEC // CHUNK
CHUNKS_PER_BATCH = PAIRS_PER_BATCH // CHUNK
ACC_PAD = 786432
ZSTRIPE = ACC_PAD // NS
ZBUF = 8192
OSTRIPE = VOX_PER_B // NS
OPIECES = [(i * ZBUF, ZBUF) for i in range(OSTRIPE // ZBUF)]
OPIECES.append((OSTRIPE - OSTRIPE % ZBUF, OSTRIPE % ZBUF))


def _tc_body(ev_ref, w1_ref, b1a_ref, w2_ref, b2_ref, w3_ref, b3_ref,
             idx_ref, val_ref):
    ev = ev_ref[0]
    x = ev[0:1, :]
    y = ev[1:2, :]
    t = ev[2:3, :]
    p = ev[3:4, :]
    b = pl.program_id(0)
    bf = b.astype(jnp.float32)
    p2 = (p + 1.0) / 2.0
    idx_before = ((x + 240.0 * y) + 388800.0 * p2) + 777600.0 * bf
    w1 = w1_ref[...]
    w2 = w2_ref[...]
    b2 = b2_ref[...]
    w3 = w3_ref[...]
    b3 = b3_ref[...]
    q = w1 * t
    for i in range(C):
        h1 = q + b1a_ref[:, i:i + 1]
        h1 = jnp.maximum(h1, NEG * h1)
        h2 = jnp.dot(w2, h1, preferred_element_type=jnp.float32) + b2
        h2 = jnp.maximum(h2, NEG * h2)
        h3 = jnp.dot(w3, h2, preferred_element_type=jnp.float32) + b3
        val = t * h3
        idxf = idx_before + float(W * H * i)
        idxg = jnp.clip(idxf, 0.0, float(NUM_VOX - 1)).astype(jnp.int32)
        lidx = idxg - b * VOX_PER_B
        lidx = jnp.clip(lidx, 0, VOX_PER_B - 1)
        idx_ref[0, i] = lidx[0]
        val_ref[0, i] = val[0]


def _tc_stage(ev_t, w1, b1, w2, b2, w3, b3):
    grid = (B, NP // CN)
    wspec = lambda shape: pl.BlockSpec(shape, lambda bb, nn: (0, 0))
    return pl.pallas_call(
        _tc_body,
        grid=grid,
        in_specs=[
            pl.BlockSpec((1, 4, CN), lambda bb, nn: (bb, 0, nn)),
            wspec((16, 1)), wspec((16, C)), wspec((16, 16)),
            wspec((16, 1)), wspec((1, 16)), wspec((1, 1)),
        ],
        out_specs=[
            pl.BlockSpec((1, C, CN), lambda bb, nn: (bb, 0, nn)),
            pl.BlockSpec((1, C, CN), lambda bb, nn: (bb, 0, nn)),
        ],
        out_shape=[
            jax.ShapeDtypeStruct((B, C, NP), jnp.int32),
            jax.ShapeDtypeStruct((B, C, NP), jnp.float32),
        ],
    )(ev_t, w1, b1, w2, b2, w3, b3)


def _sc_body(idx_hbm, val_hbm, out_hbm, acc, zbuf, idx_buf, val_buf, obuf, sem):
    c = lax.axis_index("c")
    s = lax.axis_index("s")

    def _fill(j, _):
        zbuf[pl.ds(j * 16, 16)] = jnp.zeros((16,), jnp.float32)
        return 0
    lax.fori_loop(0, ZBUF // 16, _fill, 0)

    def batch_body(bi, _):
        b = c * (B // 2) + bi
        def zero_body(z, _):
            off = pl.multiple_of(s * ZSTRIPE + z * ZBUF, 8)
            pltpu.sync_copy(zbuf, acc.at[pl.ds(off, ZBUF)])
            return 0
        lax.fori_loop(0, ZSTRIPE // ZBUF, zero_body, 0)
        plsc.subcore_barrier()

        base_cid = b * CHUNKS_PER_BATCH + s * CHUNKS_PER_TEC

        def chunk_body(ch, _):
            cid = base_cid + ch
            ld1 = pltpu.async_copy(idx_hbm.at[cid], idx_buf, sem)
            ld2 = pltpu.async_copy(val_hbm.at[cid], val_buf, sem)
            ld1.wait()
            ld2.wait()
            cps = []
            for r in range(ROWS_PER_CHUNK):
                cps.append(pltpu.async_copy(
                    val_buf.at[r], acc.at[idx_buf.at[r]], sem, add=True))
            for cp in cps:
                cp.wait()
            return 0
        lax.fori_loop(0, CHUNKS_PER_TEC, chunk_body, 0)
        plsc.subcore_barrier()

        for poff, plen in OPIECES:
            src_off = pl.multiple_of(s * OSTRIPE + poff, 8)
            dst_off = pl.multiple_of(b * VOX_PER_B + s * OSTRIPE + poff, 8)
            pltpu.sync_copy(acc.at[pl.ds(src_off, plen)],
                            obuf.at[pl.ds(0, plen)])
            pltpu.sync_copy(obuf.at[pl.ds(0, plen)],
                            out_hbm.at[pl.ds(dst_off, plen)])
        return 0

    lax.fori_loop(0, B // 2, batch_body, 0)


def _sc_scatter(idx3, val3):
    fn = pl.kernel(
        _sc_body,
        out_type=jax.ShapeDtypeStruct((NUM_VOX,), jnp.float32),
        mesh=plsc.VectorSubcoreMesh(core_axis_name="c", subcore_axis_name="s"),
        scratch_types=[
            pltpu.VMEM_SHARED((ACC_PAD,), jnp.float32),
            pltpu.VMEM((ZBUF,), jnp.float32),
            pltpu.VMEM((ROWS_PER_CHUNK, ROW), jnp.int32),
            pltpu.VMEM((ROWS_PER_CHUNK, ROW), jnp.float32),
            pltpu.VMEM((ZBUF,), jnp.float32),
            pltpu.SemaphoreType.DMA,
        ],
    )
    return fn(idx3, val3)


@jax.jit
def _run(events, w1, b1, w2, b2, w3, b3):
    ev_t = jnp.transpose(events, (0, 2, 1))
    ev_t = jnp.pad(ev_t, ((0, 0), (0, 0), (0, NP - N)))
    w1c = w1.reshape(16, 1)
    bins = (jnp.arange(C, dtype=jnp.float32) / (C - 1)).reshape(1, C)
    b1a = b1.reshape(16, 1) - w1c * bins
    idx, val = _tc_stage(
        ev_t, w1c, b1a, w2,
        b2.reshape(16, 1), w3.reshape(1, 16), b3.reshape(1, 1),
    )
    idx3 = idx.reshape(B * CHUNKS_PER_BATCH, ROWS_PER_CHUNK, ROW)
    val3 = val.reshape(B * CHUNKS_PER_BATCH, ROWS_PER_CHUNK, ROW)
    vox = _sc_scatter(idx3, val3)
    return vox.reshape(B, 2 * C, H, W)


def kernel(events_list, device, w1, b1, w2, b2, w3, b3):
    return _run(events_list, w1, b1, w2, b2, w3, b3)
